# Initial kernel scaffold; baseline (speedup 1.0000x reference)
import jax
import jax.numpy as jnp
from jax.experimental import pallas as pl


def kernel(edge_index, node_type, molecular_index):
    raise NotImplementedError("write your pallas kernel here")


# speedup vs baseline: 5.3026x; 5.3026x over previous
import functools

import jax
import jax.numpy as jnp
from jax import lax
from jax.experimental import pallas as pl
from jax.experimental.pallas import tpu as pltpu
from jax.experimental.pallas import tpu_sc as plsc

NC = 2
NS = 16
NW = NC * NS
C = 128
T = 4


def _edge_encoder(n_edges, n_nodes, src, dst, table1, table2, mol):
  nblk = n_edges // C
  mesh = plsc.VectorSubcoreMesh(core_axis_name="c", subcore_axis_name="s")

  @functools.partial(
      pl.kernel,
      out_type=jax.ShapeDtypeStruct((n_edges, T * T), jnp.float32),
      mesh=mesh,
      compiler_params=pltpu.CompilerParams(
          needs_layout_passes=False, use_tc_tiling_on_sc=False),
      scratch_types=[
          pltpu.VMEM((n_nodes,), jnp.int32),
          pltpu.VMEM((C,), jnp.int32),
          pltpu.VMEM((C,), jnp.int32),
          pltpu.VMEM((C,), jnp.int32),
          pltpu.VMEM((C, T * T), jnp.float32),
          pltpu.VMEM((C, T * T), jnp.float32),
          pltpu.SemaphoreType.DMA,
      ],
  )
  def kern(src_hbm, dst_hbm, tab1_hbm, tab2_hbm, mol_hbm, out_hbm,
           mol_v, src_v, dst_v, asrc_v, rows1_v, rows2_v, sem):
    wid = lax.axis_index("s") * NC + lax.axis_index("c")
    pltpu.sync_copy(mol_hbm, mol_v)
    zero_row = jnp.full((16,), n_nodes, jnp.int32)

    def block_body(i, carry):
      blk = wid + i * NW
      base = blk * C
      pltpu.sync_copy(src_hbm.at[pl.ds(base, C)], src_v)
      pltpu.sync_copy(dst_hbm.at[pl.ds(base, C)], dst_v)
      for j in range(C // 16):
        s16 = src_v[pl.ds(j * 16, 16)]
        d16 = dst_v[pl.ds(j * 16, 16)]
        m1 = plsc.load_gather(mol_v, [s16])
        m2 = plsc.load_gather(mol_v, [d16])
        asrc_v[pl.ds(j * 16, 16)] = jnp.where(m1 == m2, s16, zero_row)
      cp1 = pltpu.async_copy(tab1_hbm.at[asrc_v], rows1_v, sem)
      cp2 = pltpu.async_copy(tab2_hbm.at[dst_v], rows2_v, sem)
      cp1.wait()
      cp2.wait()

      def edge_body(k, carry2):
        rows1_v[k] = rows1_v[k] * rows2_v[k]
        return carry2

      lax.fori_loop(0, C, edge_body, 0, unroll=8)
      pltpu.sync_copy(rows1_v, out_hbm.at[pl.ds(base, C)])
      return carry

    trips = (nblk - wid + NW - 1) // NW
    lax.fori_loop(0, trips, block_body, 0)

  return kern(src, dst, table1, table2, mol)


@jax.jit
def kernel(edge_index, node_type, molecular_index):
  n_nodes, t = node_type.shape
  n_edges = edge_index.shape[1]
  assert t == T and n_edges % C == 0
  src = edge_index[0].astype(jnp.int32)
  dst = edge_index[1].astype(jnp.int32)
  nt = node_type.astype(jnp.float32)
  table1 = jnp.concatenate(
      [jnp.repeat(nt, T, axis=1),
       jnp.zeros((8, T * T), jnp.float32)], axis=0)
  table2 = jnp.tile(nt, (1, T))
  mol = molecular_index.astype(jnp.int32)
  return _edge_encoder(n_edges, n_nodes, src, dst, table1, table2, mol)


# speedup vs baseline: 31.1044x; 5.8658x over previous
import functools

import jax
import jax.numpy as jnp
from jax import lax
from jax.experimental import pallas as pl
from jax.experimental.pallas import tpu as pltpu
from jax.experimental.pallas import tpu_sc as plsc

NC = 2
NS = 16
NW = NC * NS
C = 128
T = 4


def _edge_encoder(n_edges, n_nodes, src, dst, table1, table2, mol):
  nblk = n_edges // C
  pairs = (nblk // NW + 2) // 2
  mesh = plsc.VectorSubcoreMesh(core_axis_name="c", subcore_axis_name="s")

  @functools.partial(
      pl.kernel,
      out_type=jax.ShapeDtypeStruct((n_edges, T * T), jnp.float32),
      mesh=mesh,
      compiler_params=pltpu.CompilerParams(
          needs_layout_passes=False, use_tc_tiling_on_sc=False),
      scratch_types=[
          pltpu.VMEM((n_nodes,), jnp.int32),
          pltpu.VMEM((C,), jnp.int32), pltpu.VMEM((C,), jnp.int32),
          pltpu.VMEM((C,), jnp.int32), pltpu.VMEM((C,), jnp.int32),
          pltpu.VMEM((C,), jnp.float32), pltpu.VMEM((C,), jnp.float32),
          pltpu.VMEM((C, T * T), jnp.float32),
          pltpu.VMEM((C, T * T), jnp.float32),
          pltpu.VMEM((C, T * T), jnp.float32),
          pltpu.VMEM((C, T * T), jnp.float32),
          pltpu.SemaphoreType.DMA, pltpu.SemaphoreType.DMA,
          pltpu.SemaphoreType.DMA, pltpu.SemaphoreType.DMA,
          pltpu.SemaphoreType.DMA, pltpu.SemaphoreType.DMA,
      ],
  )
  def kern(src_hbm, dst_hbm, tab1_hbm, tab2_hbm, mol_hbm, out_hbm,
           mol_v, src_v0, src_v1, dst_v0, dst_v1, sel_v0, sel_v1,
           r1_0, r1_1, r2_0, r2_1, si0, si1, sg0, sg1, sw0, sw1):
    src_v = (src_v0, src_v1)
    dst_v = (dst_v0, dst_v1)
    sel_v = (sel_v0, sel_v1)
    r1 = (r1_0, r1_1)
    r2 = (r2_0, r2_1)
    si = (si0, si1)
    sg = (sg0, sg1)
    sw = (sw0, sw1)

    wid = lax.axis_index("s") * NC + lax.axis_index("c")
    pltpu.sync_copy(mol_hbm, mol_v)
    ones = jnp.full((16,), 1.0, jnp.float32)
    zeros = jnp.zeros((16,), jnp.float32)

    def issue_idx(blk, u):
      base = blk * C
      pltpu.async_copy(src_hbm.at[pl.ds(base, C)], src_v[u], si[u])
      pltpu.async_copy(dst_hbm.at[pl.ds(base, C)], dst_v[u], si[u])

    def wait_idx(u):
      pltpu.make_async_copy(src_hbm.at[pl.ds(0, C)], src_v[u], si[u]).wait()
      pltpu.make_async_copy(dst_hbm.at[pl.ds(0, C)], dst_v[u], si[u]).wait()

    def wait_wb(u):
      pltpu.make_async_copy(r1[u], out_hbm.at[pl.ds(0, C)], sw[u]).wait()

    issue_idx(wid, 0)

    def pair_body(p, carry):
      for u in (0, 1):
        it = 2 * p + u
        blk = wid + it * NW

        @pl.when(blk < nblk)
        def _():
          base = blk * C
          wait_idx(u)
          nxt = blk + NW

          @pl.when(nxt < nblk)
          def _():
            issue_idx(nxt, 1 - u)

          @pl.when(p >= 1)
          def _():
            wait_wb(u)

          pltpu.async_copy(tab1_hbm.at[src_v[u]], r1[u], sg[u])
          pltpu.async_copy(tab2_hbm.at[dst_v[u]], r2[u], sg[u])

          for j in range(C // 16):
            s16 = src_v[u][pl.ds(j * 16, 16)]
            d16 = dst_v[u][pl.ds(j * 16, 16)]
            m1 = plsc.load_gather(mol_v, [s16])
            m2 = plsc.load_gather(mol_v, [d16])
            sel_v[u][pl.ds(j * 16, 16)] = jnp.where(m1 == m2, ones, zeros)

          pltpu.make_async_copy(tab1_hbm.at[src_v[u]], r1[u], sg[u]).wait()
          pltpu.make_async_copy(tab2_hbm.at[dst_v[u]], r2[u], sg[u]).wait()

          def edge_body(k, carry2):
            kk = jnp.broadcast_to(k, (16,)).astype(jnp.int32)
            s = plsc.load_gather(sel_v[u], [kk])
            r1[u][k] = r1[u][k] * r2[u][k] * s
            return carry2

          lax.fori_loop(0, C, edge_body, 0, unroll=8)
          pltpu.async_copy(r1[u], out_hbm.at[pl.ds(base, C)], sw[u])

      return carry

    lax.fori_loop(0, pairs, pair_body, 0)
    wait_wb(0)
    wait_wb(1)

  return kern(src, dst, table1, table2, mol)


@jax.jit
def kernel(edge_index, node_type, molecular_index):
  n_nodes, t = node_type.shape
  n_edges = edge_index.shape[1]
  assert t == T and n_edges % C == 0
  src = edge_index[0].astype(jnp.int32)
  dst = edge_index[1].astype(jnp.int32)
  nt = node_type.astype(jnp.float32)
  table1 = jnp.repeat(nt, T, axis=1)
  table2 = jnp.tile(nt, (1, T))
  mol = molecular_index.astype(jnp.int32)
  return _edge_encoder(n_edges, n_nodes, src, dst, table1, table2, mol)


# speedup vs baseline: 37.2110x; 1.1963x over previous
import functools

import jax
import jax.numpy as jnp
from jax import lax
from jax.experimental import pallas as pl
from jax.experimental.pallas import tpu as pltpu
from jax.experimental.pallas import tpu_sc as plsc

NC = 2
NS = 16
NW = NC * NS
S = 128
Q = 2
C = S * Q
T = 4


def _edge_encoder(n_edges, n_nodes, src2d, dst2d, table1, table2, mol):
  nblk = n_edges // C
  trips_pad = nblk // NW + 1
  mesh = plsc.VectorSubcoreMesh(core_axis_name="c", subcore_axis_name="s")

  @functools.partial(
      pl.kernel,
      out_type=jax.ShapeDtypeStruct((n_edges, T * T), jnp.float32),
      mesh=mesh,
      compiler_params=pltpu.CompilerParams(
          needs_layout_passes=False, use_tc_tiling_on_sc=False),
      scratch_types=[
          pltpu.VMEM((n_nodes,), jnp.int32),
          pltpu.VMEM((Q, S), jnp.int32), pltpu.VMEM((Q, S), jnp.int32),
          pltpu.VMEM((Q, S), jnp.int32), pltpu.VMEM((Q, S), jnp.int32),
          pltpu.VMEM((C,), jnp.float32), pltpu.VMEM((C,), jnp.float32),
          pltpu.VMEM((C, T * T), jnp.float32),
          pltpu.VMEM((C, T * T), jnp.float32),
          pltpu.VMEM((C, T * T), jnp.float32),
          pltpu.VMEM((C, T * T), jnp.float32),
          pltpu.SemaphoreType.DMA, pltpu.SemaphoreType.DMA,
          pltpu.SemaphoreType.DMA, pltpu.SemaphoreType.DMA,
          pltpu.SemaphoreType.DMA, pltpu.SemaphoreType.DMA,
      ],
  )
  def kern(src_hbm, dst_hbm, tab1_hbm, tab2_hbm, mol_hbm, out_hbm,
           mol_v, src_v0, src_v1, dst_v0, dst_v1, sel_v0, sel_v1,
           r1_0, r1_1, r2_0, r2_1, si0, si1, sg0, sg1, sw0, sw1):
    src_v = (src_v0, src_v1)
    dst_v = (dst_v0, dst_v1)
    sel_v = (sel_v0, sel_v1)
    r1 = (r1_0, r1_1)
    r2 = (r2_0, r2_1)
    si = (si0, si1)
    sg = (sg0, sg1)
    sw = (sw0, sw1)

    wid = lax.axis_index("s") * NC + lax.axis_index("c")
    pltpu.sync_copy(mol_hbm, mol_v)
    ones = jnp.full((16,), 1.0, jnp.float32)
    zeros = jnp.zeros((16,), jnp.float32)

    def issue_idx(blk, u):
      pltpu.async_copy(src_hbm.at[pl.ds(blk * Q, Q)], src_v[u], si[u])
      pltpu.async_copy(dst_hbm.at[pl.ds(blk * Q, Q)], dst_v[u], si[u])

    def wait_idx(u):
      pltpu.make_async_copy(src_hbm.at[pl.ds(0, Q)], src_v[u], si[u]).wait()
      pltpu.make_async_copy(dst_hbm.at[pl.ds(0, Q)], dst_v[u], si[u]).wait()

    def issue_gathers(u):
      for q in range(Q):
        pltpu.async_copy(tab1_hbm.at[src_v[u].at[q]],
                         r1[u].at[pl.ds(q * S, S)], sg[u])
        pltpu.async_copy(tab2_hbm.at[dst_v[u].at[q]],
                         r2[u].at[pl.ds(q * S, S)], sg[u])

    def wait_gathers(u):
      for q in range(Q):
        pltpu.make_async_copy(tab1_hbm.at[src_v[u].at[q]],
                              r1[u].at[pl.ds(q * S, S)], sg[u]).wait()
        pltpu.make_async_copy(tab2_hbm.at[dst_v[u].at[q]],
                              r2[u].at[pl.ds(q * S, S)], sg[u]).wait()

    def wait_wb(u):
      pltpu.make_async_copy(r1[u], out_hbm.at[pl.ds(0, C)], sw[u]).wait()

    issue_idx(wid, 0)
    wait_idx(0)
    issue_gathers(0)

    @pl.when(wid + NW < nblk)
    def _():
      issue_idx(wid + NW, 1)

    def iter_body(i, carry):
      u = i % 2
      blk = wid + i * NW
      for su in (0, 1):
        @pl.when((blk < nblk) & (u == su))
        def _():
          nxt = blk + NW

          @pl.when(nxt < nblk)
          def _():
            wait_idx(1 - su)

            @pl.when(i >= 1)
            def _():
              wait_wb(1 - su)

            issue_gathers(1 - su)

          for q in range(Q):
            for j in range(S // 16):
              s16 = src_v[su][q, pl.ds(j * 16, 16)]
              d16 = dst_v[su][q, pl.ds(j * 16, 16)]
              m1 = plsc.load_gather(mol_v, [s16])
              m2 = plsc.load_gather(mol_v, [d16])
              sel_v[su][pl.ds(q * S + j * 16, 16)] = (
                  jnp.where(m1 == m2, ones, zeros))

          wait_gathers(su)

          def edge_body(k, carry2):
            kk = jnp.broadcast_to(k, (16,)).astype(jnp.int32)
            s = plsc.load_gather(sel_v[su], [kk])
            r1[su][k] = r1[su][k] * r2[su][k] * s
            return carry2

          lax.fori_loop(0, C, edge_body, 0, unroll=8)
          pltpu.async_copy(r1[su], out_hbm.at[pl.ds(blk * C, C)], sw[su])

          nxt2 = blk + 2 * NW

          @pl.when(nxt2 < nblk)
          def _():
            issue_idx(nxt2, su)

      return carry

    lax.fori_loop(0, trips_pad, iter_body, 0)

    wait_wb(0)
    wait_wb(1)

  return kern(src2d, dst2d, table1, table2, mol)


@jax.jit
def kernel(edge_index, node_type, molecular_index):
  n_nodes, t = node_type.shape
  n_edges = edge_index.shape[1]
  assert t == T and n_edges % C == 0
  src2d = edge_index[0].astype(jnp.int32).reshape(n_edges // S, S)
  dst2d = edge_index[1].astype(jnp.int32).reshape(n_edges // S, S)
  nt = node_type.astype(jnp.float32)
  table1 = jnp.repeat(nt, T, axis=1)
  table2 = jnp.tile(nt, (1, T))
  mol = molecular_index.astype(jnp.int32)
  return _edge_encoder(n_edges, n_nodes, src2d, dst2d, table1, table2, mol)


# speedup vs baseline: 38.5051x; 1.0348x over previous
import functools

import jax
import jax.numpy as jnp
from jax import lax
from jax.experimental import pallas as pl
from jax.experimental.pallas import tpu as pltpu
from jax.experimental.pallas import tpu_sc as plsc

NC = 2
NS = 16
NW = NC * NS
C = 512
T = 4


def _edge_encoder(n_edges, n_nodes, src2d, dst2d, table1, table2, molp):
  nblk = n_edges // C
  trips_pad = nblk // NW + 1
  mesh = plsc.VectorSubcoreMesh(core_axis_name="c", subcore_axis_name="s")

  @functools.partial(
      pl.kernel,
      out_type=jax.ShapeDtypeStruct((n_edges, T * T), jnp.float32),
      mesh=mesh,
      compiler_params=pltpu.CompilerParams(
          needs_layout_passes=False, use_tc_tiling_on_sc=False),
      scratch_types=[
          pltpu.VMEM((n_nodes // 2,), jnp.int32),
          pltpu.VMEM((C,), jnp.int32), pltpu.VMEM((C,), jnp.int32),
          pltpu.VMEM((C,), jnp.int32), pltpu.VMEM((C,), jnp.int32),
          pltpu.VMEM((C,), jnp.float32), pltpu.VMEM((C,), jnp.float32),
          pltpu.VMEM((C, T * T), jnp.float32),
          pltpu.VMEM((C, T * T), jnp.float32),
          pltpu.VMEM((C, T * T), jnp.float32),
          pltpu.VMEM((C, T * T), jnp.float32),
          pltpu.SemaphoreType.DMA, pltpu.SemaphoreType.DMA,
          pltpu.SemaphoreType.DMA, pltpu.SemaphoreType.DMA,
          pltpu.SemaphoreType.DMA, pltpu.SemaphoreType.DMA,
      ],
  )
  def kern(src_hbm, dst_hbm, tab1_hbm, tab2_hbm, molp_hbm, out_hbm,
           molp_v, src_v0, src_v1, dst_v0, dst_v1, sel_v0, sel_v1,
           r1_0, r1_1, r2_0, r2_1, si0, si1, sg0, sg1, sw0, sw1):
    src_v = (src_v0, src_v1)
    dst_v = (dst_v0, dst_v1)
    sel_v = (sel_v0, sel_v1)
    r1 = (r1_0, r1_1)
    r2 = (r2_0, r2_1)
    si = (si0, si1)
    sg = (sg0, sg1)
    sw = (sw0, sw1)

    wid = lax.axis_index("s") * NC + lax.axis_index("c")
    pltpu.sync_copy(molp_hbm, molp_v)
    ones = jnp.full((16,), 1.0, jnp.float32)
    zeros = jnp.zeros((16,), jnp.float32)

    def issue_idx(blk, u):
      pltpu.async_copy(src_hbm.at[blk], src_v[u], si[u])
      pltpu.async_copy(dst_hbm.at[blk], dst_v[u], si[u])

    def wait_idx(u):
      pltpu.make_async_copy(src_hbm.at[0], src_v[u], si[u]).wait()
      pltpu.make_async_copy(dst_hbm.at[0], dst_v[u], si[u]).wait()

    def issue_gathers(u):
      pltpu.async_copy(tab1_hbm.at[src_v[u]], r1[u], sg[u])
      pltpu.async_copy(tab2_hbm.at[dst_v[u]], r2[u], sg[u])

    def wait_gathers(u):
      pltpu.make_async_copy(tab1_hbm.at[src_v[u]], r1[u], sg[u]).wait()
      pltpu.make_async_copy(tab2_hbm.at[dst_v[u]], r2[u], sg[u]).wait()

    def wait_wb(u):
      pltpu.make_async_copy(r1[u], out_hbm.at[pl.ds(0, C)], sw[u]).wait()

    def mol_at(idx16):
      w = plsc.load_gather(molp_v, [idx16 >> 1])
      return (w >> ((idx16 & 1) << 4)) & 0xFFFF

    issue_idx(wid, 0)
    wait_idx(0)
    issue_gathers(0)

    @pl.when(wid + NW < nblk)
    def _():
      issue_idx(wid + NW, 1)

    def iter_body(i, carry):
      u = i % 2
      blk = wid + i * NW
      for su in (0, 1):
        @pl.when((blk < nblk) & (u == su))
        def _():
          nxt = blk + NW

          @pl.when(nxt < nblk)
          def _():
            wait_idx(1 - su)

            @pl.when(i >= 1)
            def _():
              wait_wb(1 - su)

            issue_gathers(1 - su)

          for j in range(C // 16):
            s16 = src_v[su][pl.ds(j * 16, 16)]
            d16 = dst_v[su][pl.ds(j * 16, 16)]
            sel_v[su][pl.ds(j * 16, 16)] = (
                jnp.where(mol_at(s16) == mol_at(d16), ones, zeros))

          wait_gathers(su)

          def edge_body(k, carry2):
            kk = jnp.broadcast_to(k, (16,)).astype(jnp.int32)
            s = plsc.load_gather(sel_v[su], [kk])
            r1[su][k] = r1[su][k] * r2[su][k] * s
            return carry2

          lax.fori_loop(0, C, edge_body, 0, unroll=8)
          pltpu.async_copy(r1[su], out_hbm.at[pl.ds(blk * C, C)], sw[su])

          nxt2 = blk + 2 * NW

          @pl.when(nxt2 < nblk)
          def _():
            issue_idx(nxt2, su)

      return carry

    lax.fori_loop(0, trips_pad, iter_body, 0)

    wait_wb(0)
    wait_wb(1)

  return kern(src2d, dst2d, table1, table2, molp)


@jax.jit
def kernel(edge_index, node_type, molecular_index):
  n_nodes, t = node_type.shape
  n_edges = edge_index.shape[1]
  assert t == T and n_edges % C == 0 and n_nodes % 2 == 0
  src2d = edge_index[0].astype(jnp.int32).reshape(n_edges // C, C)
  dst2d = edge_index[1].astype(jnp.int32).reshape(n_edges // C, C)
  nt = node_type.astype(jnp.float32)
  table1 = jnp.repeat(nt, T, axis=1)
  table2 = jnp.tile(nt, (1, T))
  mol = molecular_index.astype(jnp.int32)
  molp = mol[0::2] | (mol[1::2] << 16)
  return _edge_encoder(n_edges, n_nodes, src2d, dst2d, table1, table2, molp)


# speedup vs baseline: 38.8750x; 1.0096x over previous
import functools

import jax
import jax.numpy as jnp
from jax import lax
from jax.experimental import pallas as pl
from jax.experimental.pallas import tpu as pltpu
from jax.experimental.pallas import tpu_sc as plsc

NC = 2
NS = 16
NW = NC * NS
C = 512
T = 4


def _edge_encoder(n_edges, n_nodes, edge_idx, table1, table2, molp):
  nblk = n_edges // C
  trips_pad = nblk // NW + 1
  mesh = plsc.VectorSubcoreMesh(core_axis_name="c", subcore_axis_name="s")

  @functools.partial(
      pl.kernel,
      out_type=jax.ShapeDtypeStruct((n_edges, T * T), jnp.float32),
      mesh=mesh,
      compiler_params=pltpu.CompilerParams(
          needs_layout_passes=False, use_tc_tiling_on_sc=False),
      scratch_types=[
          pltpu.VMEM((n_nodes // 2,), jnp.int32),
          pltpu.VMEM((C,), jnp.int32), pltpu.VMEM((C,), jnp.int32),
          pltpu.VMEM((C,), jnp.int32), pltpu.VMEM((C,), jnp.int32),
          pltpu.VMEM((C,), jnp.float32), pltpu.VMEM((C,), jnp.float32),
          pltpu.VMEM((C, T * T), jnp.float32),
          pltpu.VMEM((C, T * T), jnp.float32),
          pltpu.VMEM((C, T * T), jnp.float32),
          pltpu.VMEM((C, T * T), jnp.float32),
          pltpu.SemaphoreType.DMA, pltpu.SemaphoreType.DMA,
          pltpu.SemaphoreType.DMA, pltpu.SemaphoreType.DMA,
          pltpu.SemaphoreType.DMA, pltpu.SemaphoreType.DMA,
      ],
  )
  def kern(edge_hbm, tab1_hbm, tab2_hbm, molp_hbm, out_hbm,
           molp_v, src_v0, src_v1, dst_v0, dst_v1, sel_v0, sel_v1,
           r1_0, r1_1, r2_0, r2_1, si0, si1, sg0, sg1, sw0, sw1):
    src_v = (src_v0, src_v1)
    dst_v = (dst_v0, dst_v1)
    sel_v = (sel_v0, sel_v1)
    r1 = (r1_0, r1_1)
    r2 = (r2_0, r2_1)
    si = (si0, si1)
    sg = (sg0, sg1)
    sw = (sw0, sw1)

    wid = lax.axis_index("s") * NC + lax.axis_index("c")
    pltpu.sync_copy(molp_hbm, molp_v)
    ones = jnp.full((16,), 1.0, jnp.float32)
    zeros = jnp.zeros((16,), jnp.float32)

    def issue_idx(blk, u):
      pltpu.async_copy(edge_hbm.at[0, pl.ds(blk * C, C)], src_v[u], si[u])
      pltpu.async_copy(edge_hbm.at[1, pl.ds(blk * C, C)], dst_v[u], si[u])

    def wait_idx(u):
      pltpu.make_async_copy(edge_hbm.at[0, pl.ds(0, C)], src_v[u], si[u]).wait()
      pltpu.make_async_copy(edge_hbm.at[1, pl.ds(0, C)], dst_v[u], si[u]).wait()

    def issue_gathers(u):
      pltpu.async_copy(tab1_hbm.at[src_v[u]], r1[u], sg[u])
      pltpu.async_copy(tab2_hbm.at[dst_v[u]], r2[u], sg[u])

    def wait_gathers(u):
      pltpu.make_async_copy(tab1_hbm.at[src_v[u]], r1[u], sg[u]).wait()
      pltpu.make_async_copy(tab2_hbm.at[dst_v[u]], r2[u], sg[u]).wait()

    def wait_wb(u):
      pltpu.make_async_copy(r1[u], out_hbm.at[pl.ds(0, C)], sw[u]).wait()

    def mol_at(idx16):
      w = plsc.load_gather(molp_v, [idx16 >> 1])
      return (w >> ((idx16 & 1) << 4)) & 0xFFFF

    issue_idx(wid, 0)
    wait_idx(0)
    issue_gathers(0)

    @pl.when(wid + NW < nblk)
    def _():
      issue_idx(wid + NW, 1)

    def iter_body(i, carry):
      u = i % 2
      blk = wid + i * NW
      for su in (0, 1):
        @pl.when((blk < nblk) & (u == su))
        def _():
          nxt = blk + NW

          @pl.when(nxt < nblk)
          def _():
            wait_idx(1 - su)

            @pl.when(i >= 1)
            def _():
              wait_wb(1 - su)

            issue_gathers(1 - su)

          for j in range(C // 16):
            s16 = src_v[su][pl.ds(j * 16, 16)]
            d16 = dst_v[su][pl.ds(j * 16, 16)]
            sel_v[su][pl.ds(j * 16, 16)] = (
                jnp.where(mol_at(s16) == mol_at(d16), ones, zeros))

          wait_gathers(su)

          def edge_body(k, carry2):
            kk = jnp.broadcast_to(k, (16,)).astype(jnp.int32)
            s = plsc.load_gather(sel_v[su], [kk])
            r1[su][k] = r1[su][k] * r2[su][k] * s
            return carry2

          lax.fori_loop(0, C, edge_body, 0, unroll=8)
          pltpu.async_copy(r1[su], out_hbm.at[pl.ds(blk * C, C)], sw[su])

          nxt2 = blk + 2 * NW

          @pl.when(nxt2 < nblk)
          def _():
            issue_idx(nxt2, su)

      return carry

    lax.fori_loop(0, trips_pad, iter_body, 0)

    wait_wb(0)
    wait_wb(1)

  return kern(edge_idx, table1, table2, molp)


@jax.jit
def kernel(edge_index, node_type, molecular_index):
  n_nodes, t = node_type.shape
  n_edges = edge_index.shape[1]
  assert t == T and n_edges % C == 0 and n_nodes % 2 == 0
  edge_idx = edge_index.astype(jnp.int32)
  nt = node_type.astype(jnp.float32)
  table1 = jnp.repeat(nt, T, axis=1)
  table2 = jnp.tile(nt, (1, T))
  mol = molecular_index.astype(jnp.int32)
  molp = mol[0::2] | (mol[1::2] << 16)
  return _edge_encoder(n_edges, n_nodes, edge_idx, table1, table2, molp)
